# baseline (speedup 1.0000x reference)
import functools

import jax
import jax.numpy as jnp
from jax import lax
from jax.experimental import pallas as pl
from jax.experimental.pallas import tpu as pltpu
from jax.experimental.pallas import tpu_sc as plsc

K = 1024
B = 32
NW = 32
PAD = 2047


def _vq_body(x_ref, c_ref, o_ref):
    x = x_ref[...]
    c = c_ref[...]
    d = (x - c) ** 2
    m = jnp.min(d, axis=1, keepdims=True)
    lane = lax.broadcasted_iota(jnp.int32, d.shape, 1)
    o_ref[...] = jnp.min(jnp.where(d == m, lane, K), axis=1, keepdims=True)


def _vq_assign(img_flat, centroids_row):
    n = img_flat.shape[0]
    blk = n // 32
    return pl.pallas_call(
        _vq_body,
        grid=(32,),
        in_specs=[
            pl.BlockSpec((blk, 1), lambda i: (i, 0)),
            pl.BlockSpec((1, K), lambda i: (0, 0)),
        ],
        out_specs=pl.BlockSpec((blk, 1), lambda i: (i, 0)),
        out_shape=jax.ShapeDtypeStruct((n, 1), jnp.int32),
    )(img_flat, centroids_row)


def _bitonic_body(x_ref, o_ref, *, seg):
    v = x_ref[...]
    w = v.shape[1]
    pos = lax.broadcasted_iota(jnp.int32, (1, w), 1)
    ids = pos & (seg - 1)
    k = 2
    while k <= seg:
        j = k // 2
        while j >= 1:
            up = jnp.concatenate([v[:, j:], v[:, :j]], axis=1)
            down = jnp.concatenate([v[:, w - j:], v[:, :w - j]], axis=1)
            bit = (ids & j) != 0
            partner = jnp.where(bit, down, up)
            dirn = (ids & k) == 0
            take_min = jnp.logical_xor(bit, dirn)
            mn = jnp.minimum(v, partner)
            mx = jnp.maximum(v, partner)
            v = jnp.where(take_min, mn, mx)
            j //= 2
        k *= 2
    o_ref[...] = v


def _bitonic_sort(x, seg, grid):
    n, w = x.shape
    blk = n // grid
    return pl.pallas_call(
        functools.partial(_bitonic_body, seg=seg),
        grid=(grid,),
        in_specs=[pl.BlockSpec((blk, w), lambda i: (i, 0))],
        out_specs=pl.BlockSpec((blk, w), lambda i: (i, 0)),
        out_shape=jax.ShapeDtypeStruct((n, w), jnp.int32),
    )(x)


def _fc_body(f_ref, w1_ref, w2_ref, w3_ref, o_ref):
    dn = (((1,), (1,)), ((), ()))
    kw = dict(precision=lax.Precision.HIGHEST, preferred_element_type=jnp.float32)
    h = jnp.maximum(lax.dot_general(f_ref[...], w1_ref[...], dn, **kw), 0.0)
    h = jnp.maximum(lax.dot_general(h, w2_ref[...], dn, **kw), 0.0)
    lg = lax.dot_general(h, w3_ref[...], dn, **kw)
    m = jnp.max(lg, axis=1, keepdims=True)
    e = jnp.exp(lg - m)
    o_ref[...] = e / jnp.sum(e, axis=1, keepdims=True)


def _fc_tail(feat, w1, w2, w3):
    return pl.pallas_call(
        _fc_body,
        out_shape=jax.ShapeDtypeStruct((feat.shape[0], 10), jnp.float32),
    )(feat, w1, w2, w3)


_MESH = dict(core_axis_name="c", subcore_axis_name="s")


def _wid():
    return lax.axis_index("s") * 2 + lax.axis_index("c")


def _chunk_of(n_per):
    return 128 if n_per % 128 == 0 else 96


def _group_of(nch):
    for g in (10, 9, 8, 6, 5, 4, 3, 2, 1):
        if nch % g == 0:
            return g


def _sc_gather(table, idx):
    n = idx.shape[0]
    n_per = n // NW
    ck = _chunk_of(n_per)
    nch = n_per // ck
    group = _group_of(nch)
    ngroups = nch // group
    mesh = plsc.VectorSubcoreMesh(**_MESH)

    @functools.partial(
        pl.kernel, mesh=mesh,
        out_type=jax.ShapeDtypeStruct((n,), jnp.int32),
        scratch_types=[
            pltpu.VMEM((n_per,), jnp.int32),
            pltpu.VMEM((n_per,), jnp.int32),
            pltpu.SemaphoreType.DMA,
        ],
        compiler_params=pltpu.CompilerParams(needs_layout_passes=False),
    )
    def k(table_hbm, idx_hbm, out_hbm, idx_v, out_v, sem):
        base = _wid() * n_per
        pltpu.sync_copy(idx_hbm.at[pl.ds(base, n_per)], idx_v)

        def grp(gi, carry):
            off0 = gi * (group * ck)
            hs = []
            for b in range(group):
                off = off0 + b * ck
                hs.append(pltpu.async_copy(
                    table_hbm.at[idx_v.at[pl.ds(off, ck)]],
                    out_v.at[pl.ds(off, ck)], sem))
            for h in hs:
                h.wait()
            return carry

        lax.fori_loop(0, ngroups, grp, 0)
        pltpu.sync_copy(out_v, out_hbm.at[pl.ds(base, n_per)])

    return k(table, idx)


def _sc_fold(svals, add_flat, relu_lut, cent_flat, n_chains, s_pad, s_real,
             decode):
    n_per = n_chains // NW
    ck = _chunk_of(n_per)
    nch = n_per // ck
    nv = n_per // 16
    mesh = plsc.VectorSubcoreMesh(**_MESH)
    out_dtype = jnp.float32 if decode else jnp.int32

    @functools.partial(
        pl.kernel, mesh=mesh,
        out_type=jax.ShapeDtypeStruct((n_chains,), out_dtype),
        scratch_types=[
            pltpu.VMEM((n_per * s_pad,), jnp.int32),
            pltpu.VMEM((n_per,), jnp.int32),
            pltpu.VMEM((n_per,), jnp.int32),
            pltpu.VMEM((n_per,), out_dtype),
            pltpu.VMEM((K,), jnp.int32),
            pltpu.VMEM((K,), jnp.float32),
            pltpu.SemaphoreType.DMA,
        ],
        compiler_params=pltpu.CompilerParams(needs_layout_passes=False),
    )
    def k(sv_hbm, add_hbm, relu_hbm, cent_hbm, out_hbm, vals_v, tmp_v, idx_v,
          res_v, relu_v, cent_v, sem):
        base = _wid() * (n_per * s_pad)
        pltpu.sync_copy(sv_hbm.at[pl.ds(base, n_per * s_pad)], vals_v)
        pltpu.sync_copy(relu_hbm, relu_v)
        if decode:
            pltpu.sync_copy(cent_hbm, cent_v)

        lane_off = lax.iota(jnp.int32, 16) * s_pad

        for i in range(nv):
            tmp_v[pl.ds(i * 16, 16)] = plsc.load_gather(
                vals_v, [lane_off + (i * 16 * s_pad)])

        def step(j, carry):
            for i in range(nv):
                vj = plsc.load_gather(vals_v, [lane_off + (i * 16 * s_pad + j)])
                idx_v[pl.ds(i * 16, 16)] = vj * K + tmp_v[pl.ds(i * 16, 16)]
            hs = [pltpu.async_copy(
                add_hbm.at[idx_v.at[pl.ds(c * ck, ck)]],
                tmp_v.at[pl.ds(c * ck, ck)], sem) for c in range(nch)]
            for h in hs:
                h.wait()
            return carry

        lax.fori_loop(1, s_real, step, 0)

        for i in range(nv):
            sl = pl.ds(i * 16, 16)
            sym = plsc.load_gather(relu_v, [tmp_v[sl]])
            if decode:
                res_v[sl] = plsc.load_gather(cent_v, [sym])
            else:
                res_v[sl] = sym
        pltpu.sync_copy(res_v, out_hbm.at[pl.ds(_wid() * n_per, n_per)])

    return k(svals, add_flat, relu_lut, cent_flat)


def _windows_2d(sym):
    b = sym.shape[0]
    taps = []
    for ki in range(5):
        for kj in range(5):
            taps.append(sym[:, ki:ki + 23:2, kj:kj + 23:2])
    p = jnp.stack(taps, axis=3)
    return p.reshape(b, 144, 25)


def _windows_3d(c1):
    b = c1.shape[0]
    taps = []
    for ki in range(5):
        for kj in range(5):
            taps.append(c1[:, ki:ki + 7:2, kj:kj + 7:2, :])
    p = jnp.stack(taps, axis=3)
    p = jnp.transpose(p, (0, 1, 2, 4, 3))
    return p.reshape(b, 16, 200)


def _forward(x, conv_flat, add_flat, cent_flat, cent_row, W1, W2, W3,
             c1_weights, c2_weights, relu_lut):
    b = x.shape[0]
    nc1 = b * 144 * 8
    nc2 = b * 16 * 16

    sym = _vq_assign(x.reshape(b * 784, 1), cent_row).reshape(b, 28, 28)

    p1 = _windows_2d(sym)
    idx1 = (p1[:, :, None, :] * K
            + c1_weights.T[None, None, :, :]).reshape(-1)
    vals1 = _sc_gather(conv_flat, idx1)

    v1 = vals1.reshape(nc1, 25)
    v1 = jnp.pad(v1, ((0, 0), (0, 7)), constant_values=PAD)
    s1 = _bitonic_sort(v1.reshape(nc1 // 4, 128), seg=32, grid=8)

    c1 = _sc_fold(s1.reshape(-1), add_flat, relu_lut, cent_flat, nc1, 32,
                  25, decode=False)
    c1 = c1.reshape(b, 12, 12, 8)

    p2 = _windows_3d(c1)
    idx2 = (p2[:, :, None, :] * K
            + c2_weights.T[None, None, :, :]).reshape(-1)
    vals2 = _sc_gather(conv_flat, idx2)

    v2 = vals2.reshape(nc2, 200)
    v2 = jnp.pad(v2, ((0, 0), (0, 56)), constant_values=PAD)
    s2 = _bitonic_sort(v2, seg=256, grid=8)

    dec = _sc_fold(s2.reshape(-1), add_flat, relu_lut, cent_flat, nc2, 256,
                   200, decode=True)

    feat = jnp.transpose(dec.reshape(b, 4, 4, 16), (0, 3, 1, 2)).reshape(b, 256)
    return _fc_tail(feat, W1, W2, W3)


def kernel(x_bat, centroid_lut, W1, W2, W3, c1_weights, c2_weights, conv_lut,
           add_lut, relu_lut):
    conv_flat = conv_lut.reshape(-1)
    add_flat = add_lut.reshape(-1)
    cent_flat = centroid_lut.reshape(-1)
    cent_row = centroid_lut.reshape(1, K)
    args = (conv_flat, add_flat, cent_flat, cent_row, W1, W2, W3,
            c1_weights, c2_weights, relu_lut)
    h = B // 2
    lo = _forward(x_bat[:h], *args)
    hi = _forward(x_bat[h:], *args)
    return jnp.concatenate([lo, hi], axis=0)


# speedup vs baseline: 1.1898x; 1.1898x over previous
import functools

import jax
import jax.numpy as jnp
from jax import lax
from jax.experimental import pallas as pl
from jax.experimental.pallas import tpu as pltpu
from jax.experimental.pallas import tpu_sc as plsc

K = 1024
B = 32
NW = 32
PAD = 2047


def _vq_body(x_ref, c_ref, o_ref):
    x = x_ref[...]
    c = c_ref[...]
    d = (x - c) ** 2
    m = jnp.min(d, axis=1, keepdims=True)
    lane = lax.broadcasted_iota(jnp.int32, d.shape, 1)
    o_ref[...] = jnp.min(jnp.where(d == m, lane, K), axis=1, keepdims=True)


def _vq_assign(img_flat, centroids_row):
    n = img_flat.shape[0]
    blk = n // 32
    return pl.pallas_call(
        _vq_body,
        grid=(32,),
        in_specs=[
            pl.BlockSpec((blk, 1), lambda i: (i, 0)),
            pl.BlockSpec((1, K), lambda i: (0, 0)),
        ],
        out_specs=pl.BlockSpec((blk, 1), lambda i: (i, 0)),
        out_shape=jax.ShapeDtypeStruct((n, 1), jnp.int32),
    )(img_flat, centroids_row)


def _bitonic_body(x_ref, o_ref, *, seg):
    v = x_ref[...]
    w = v.shape[1]
    pos = lax.broadcasted_iota(jnp.int32, (1, w), 1)
    ids = pos & (seg - 1)
    k = 2
    while k <= seg:
        j = k // 2
        while j >= 1:
            up = jnp.concatenate([v[:, j:], v[:, :j]], axis=1)
            down = jnp.concatenate([v[:, w - j:], v[:, :w - j]], axis=1)
            bit = (ids & j) != 0
            partner = jnp.where(bit, down, up)
            dirn = (ids & k) == 0
            take_min = jnp.logical_xor(bit, dirn)
            mn = jnp.minimum(v, partner)
            mx = jnp.maximum(v, partner)
            v = jnp.where(take_min, mn, mx)
            j //= 2
        k *= 2
    o_ref[...] = v


def _bitonic_sort(x, seg, grid):
    n, w = x.shape
    blk = n // grid
    return pl.pallas_call(
        functools.partial(_bitonic_body, seg=seg),
        grid=(grid,),
        in_specs=[pl.BlockSpec((blk, w), lambda i: (i, 0))],
        out_specs=pl.BlockSpec((blk, w), lambda i: (i, 0)),
        out_shape=jax.ShapeDtypeStruct((n, w), jnp.int32),
    )(x)


def _fc_body(f_ref, w1_ref, w2_ref, w3_ref, o_ref):
    dn = (((1,), (1,)), ((), ()))
    kw = dict(precision=lax.Precision.HIGHEST, preferred_element_type=jnp.float32)
    h = jnp.maximum(lax.dot_general(f_ref[...], w1_ref[...], dn, **kw), 0.0)
    h = jnp.maximum(lax.dot_general(h, w2_ref[...], dn, **kw), 0.0)
    lg = lax.dot_general(h, w3_ref[...], dn, **kw)
    m = jnp.max(lg, axis=1, keepdims=True)
    e = jnp.exp(lg - m)
    o_ref[...] = e / jnp.sum(e, axis=1, keepdims=True)


def _fc_tail(feat, w1, w2, w3):
    return pl.pallas_call(
        _fc_body,
        out_shape=jax.ShapeDtypeStruct((feat.shape[0], 10), jnp.float32),
    )(feat, w1, w2, w3)


_MESH = dict(core_axis_name="c", subcore_axis_name="s")


def _wid():
    return lax.axis_index("s") * 2 + lax.axis_index("c")


def _chunk_of(n_per):
    return 128 if n_per % 128 == 0 else 96


def _group_of(nch):
    for g in (10, 9, 8, 6, 5, 4, 3, 2, 1):
        if nch % g == 0:
            return g


def _sc_gather(table, idx):
    n = idx.shape[0]
    n_per = n // NW
    ck = _chunk_of(n_per)
    nch = n_per // ck
    group = _group_of(nch)
    ngroups = nch // group
    mesh = plsc.VectorSubcoreMesh(**_MESH)

    @functools.partial(
        pl.kernel, mesh=mesh,
        out_type=jax.ShapeDtypeStruct((n,), jnp.int32),
        scratch_types=[
            pltpu.VMEM((n_per,), jnp.int32),
            pltpu.VMEM((n_per,), jnp.int32),
            pltpu.SemaphoreType.DMA,
        ],
        compiler_params=pltpu.CompilerParams(needs_layout_passes=False),
    )
    def k(table_hbm, idx_hbm, out_hbm, idx_v, out_v, sem):
        base = _wid() * n_per
        pltpu.sync_copy(idx_hbm.at[pl.ds(base, n_per)], idx_v)

        def grp(gi, carry):
            off0 = gi * (group * ck)
            hs = []
            for b in range(group):
                off = off0 + b * ck
                hs.append(pltpu.async_copy(
                    table_hbm.at[idx_v.at[pl.ds(off, ck)]],
                    out_v.at[pl.ds(off, ck)], sem))
            for h in hs:
                h.wait()
            return carry

        lax.fori_loop(0, ngroups, grp, 0)
        pltpu.sync_copy(out_v, out_hbm.at[pl.ds(base, n_per)])

    return k(table, idx)


def _sc_fold(svals, add_flat, relu_lut, cent_flat, n_chains, s_pad, s_real,
             decode):
    n_per = n_chains // NW
    ck = _chunk_of(n_per)
    nch = n_per // ck
    nv = n_per // 16
    mesh = plsc.VectorSubcoreMesh(**_MESH)
    out_dtype = jnp.float32 if decode else jnp.int32

    @functools.partial(
        pl.kernel, mesh=mesh,
        out_type=jax.ShapeDtypeStruct((n_chains,), out_dtype),
        scratch_types=[
            pltpu.VMEM((n_per * s_pad,), jnp.int32),
            pltpu.VMEM((n_per,), jnp.int32),
            pltpu.VMEM((n_per,), jnp.int32),
            pltpu.VMEM((n_per,), out_dtype),
            pltpu.VMEM((K,), jnp.int32),
            pltpu.VMEM((K,), jnp.float32),
            pltpu.SemaphoreType.DMA,
        ],
        compiler_params=pltpu.CompilerParams(needs_layout_passes=False),
    )
    def k(sv_hbm, add_hbm, relu_hbm, cent_hbm, out_hbm, vals_v, tmp_v, idx_v,
          res_v, relu_v, cent_v, sem):
        base = _wid() * (n_per * s_pad)
        pltpu.sync_copy(sv_hbm.at[pl.ds(base, n_per * s_pad)], vals_v)
        pltpu.sync_copy(relu_hbm, relu_v)
        if decode:
            pltpu.sync_copy(cent_hbm, cent_v)

        lane_off = lax.iota(jnp.int32, 16) * s_pad

        for i in range(nv):
            tmp_v[pl.ds(i * 16, 16)] = plsc.load_gather(
                vals_v, [lane_off + (i * 16 * s_pad)])

        def step(j, carry):
            for i in range(nv):
                vj = plsc.load_gather(vals_v, [lane_off + (i * 16 * s_pad + j)])
                idx_v[pl.ds(i * 16, 16)] = vj * K + tmp_v[pl.ds(i * 16, 16)]
            hs = [pltpu.async_copy(
                add_hbm.at[idx_v.at[pl.ds(c * ck, ck)]],
                tmp_v.at[pl.ds(c * ck, ck)], sem) for c in range(nch)]
            for h in hs:
                h.wait()
            return carry

        lax.fori_loop(1, s_real, step, 0)

        for i in range(nv):
            sl = pl.ds(i * 16, 16)
            sym = plsc.load_gather(relu_v, [tmp_v[sl]])
            if decode:
                res_v[sl] = plsc.load_gather(cent_v, [sym])
            else:
                res_v[sl] = sym
        pltpu.sync_copy(res_v, out_hbm.at[pl.ds(_wid() * n_per, n_per)])

    return k(svals, add_flat, relu_lut, cent_flat)


def _windows_2d(sym):
    b = sym.shape[0]
    taps = []
    for ki in range(5):
        for kj in range(5):
            taps.append(sym[:, ki:ki + 23:2, kj:kj + 23:2])
    p = jnp.stack(taps, axis=3)
    return p.reshape(b, 144, 25)


def _windows_3d(c1):
    b = c1.shape[0]
    taps = []
    for ki in range(5):
        for kj in range(5):
            taps.append(c1[:, ki:ki + 7:2, kj:kj + 7:2, :])
    p = jnp.stack(taps, axis=3)
    p = jnp.transpose(p, (0, 1, 2, 4, 3))
    return p.reshape(b, 16, 200)


def _forward(x, conv_flat, add_flat, cent_flat, cent_row, W1, W2, W3,
             c1_weights, c2_weights, relu_lut):
    b = x.shape[0]
    nc1 = b * 144 * 8
    nc2 = b * 16 * 16

    sym = _vq_assign(x.reshape(b * 784, 1), cent_row).reshape(b, 28, 28)

    p1 = _windows_2d(sym)
    idx1 = (p1[:, :, None, :] * K
            + c1_weights.T[None, None, :, :]).reshape(-1)
    vals1 = _sc_gather(conv_flat, idx1)

    v1 = vals1.reshape(nc1, 25)
    v1 = jnp.pad(v1, ((0, 0), (0, 7)), constant_values=PAD)
    s1 = _bitonic_sort(v1.reshape(nc1 // 4, 128), seg=32, grid=8)

    c1 = _sc_fold(s1.reshape(-1), add_flat, relu_lut, cent_flat, nc1, 32,
                  25, decode=False)
    c1 = c1.reshape(b, 12, 12, 8)

    p2 = _windows_3d(c1)
    idx2 = (p2[:, :, None, :] * K
            + c2_weights.T[None, None, :, :]).reshape(-1)
    vals2 = _sc_gather(conv_flat, idx2)

    v2 = vals2.reshape(nc2, 200)
    v2 = jnp.pad(v2, ((0, 0), (0, 56)), constant_values=PAD)
    s2 = _bitonic_sort(v2, seg=256, grid=8)

    dec = _sc_fold(s2.reshape(-1), add_flat, relu_lut, cent_flat, nc2, 256,
                   200, decode=True)

    feat = jnp.transpose(dec.reshape(b, 4, 4, 16), (0, 3, 1, 2)).reshape(b, 256)
    return _fc_tail(feat, W1, W2, W3)


def kernel(x_bat, centroid_lut, W1, W2, W3, c1_weights, c2_weights, conv_lut,
           add_lut, relu_lut):
    conv_flat = conv_lut.reshape(-1)
    add_flat = add_lut.reshape(-1)
    cent_flat = centroid_lut.reshape(-1)
    cent_row = centroid_lut.reshape(1, K)
    args = (conv_flat, add_flat, cent_flat, cent_row, W1, W2, W3,
            c1_weights, c2_weights, relu_lut)
    return _forward(x_bat, *args)
